# baseline (speedup 1.0000x reference)
import functools

import jax
import jax.numpy as jnp
from jax import lax
from jax.experimental import pallas as pl
from jax.experimental.pallas import tpu as pltpu
from jax.experimental.pallas import tpu_sc as plsc

_PAD = 1
_LANES = 16
_NW = 32
_K = 32


@functools.lru_cache(maxsize=None)
def _build_sc_kernel(B, S, D):
    SPAN = S // _NW
    NCHUNK = SPAN // _K
    mesh = plsc.VectorSubcoreMesh(core_axis_name="c", subcore_axis_name="s")

    @functools.partial(
        pl.kernel,
        out_type=jax.ShapeDtypeStruct((B * S, D), jnp.float32),
        mesh=mesh,
        scratch_types=[
            pltpu.VMEM((S,), jnp.int32),
            pltpu.VMEM((S,), jnp.int32),
            pltpu.VMEM((B * SPAN,), jnp.int32),
            pltpu.VMEM((_K, D), jnp.float32),
            pltpu.VMEM((_K, D), jnp.float32),
            pltpu.VMEM((_K, D), jnp.float32),
            pltpu.SemaphoreType.DMA,
            pltpu.SemaphoreType.DMA,
            pltpu.SemaphoreType.DMA,
            pltpu.SemaphoreType.DMA,
            pltpu.SemaphoreType.DMA,
            pltpu.SemaphoreType.DMA,
        ],
        compiler_params=pltpu.CompilerParams(needs_layout_passes=False),
    )
    def sc_kernel(
        ids_hbm, table_hbm, out_hbm,
        ids0_v, ids1_v, idx_v, bufa0, bufa1, bufb,
        gsem0, gsem1, osem0, osem1, msem, isem,
    ):
        wid = lax.axis_index("s") * 2 + lax.axis_index("c")
        tok0 = wid * SPAN

        one = jnp.full((_LANES,), 1, jnp.int32)
        zero = jnp.full((_LANES,), 0, jnp.int32)
        pad_vec = jnp.full((_LANES,), _PAD, jnp.int32)

        _UNROLL = 4
        pltpu.async_copy(ids_hbm.at[pl.ds(0, S)], ids0_v, isem)
        for r in range(B):
            ids_v = ids0_v if r % 2 == 0 else ids1_v
            nxt_v = ids1_v if r % 2 == 0 else ids0_v
            pltpu.make_async_copy(ids_hbm.at[pl.ds(0, S)], ids_v, isem).wait()
            if r + 1 < B:
                pltpu.async_copy(
                    ids_hbm.at[pl.ds((r + 1) * S, S)], nxt_v, isem
                )

            def _cnt(j, acc, ids_v=ids_v):
                for u in range(_UNROLL):
                    v = ids_v[pl.ds((j * _UNROLL + u) * _LANES, _LANES)]
                    acc = acc + jnp.where(v != _PAD, one, zero)
                return acc
            accv = lax.fori_loop(
                0, wid * (SPAN // (_UNROLL * _LANES)), _cnt, zero
            )
            prefix = jnp.sum(accv)

            def _pos(j, run, ids_v=ids_v, r=r):
                v = ids_v[pl.ds(tok0 + j * _LANES, _LANES)]
                m = v != _PAD
                c = jnp.cumsum(jnp.where(m, one, zero))
                idx_v[pl.ds(r * SPAN + j * _LANES, _LANES)] = jnp.where(
                    m, c + run, pad_vec
                )
                return run + c[_LANES - 1]
            lax.fori_loop(0, SPAN // _LANES, _pos, prefix + jnp.int32(1))

        bufs = (bufa0, bufa1)
        gsems = (gsem0, gsem1)
        osems = (osem0, osem1)
        cnt = [jnp.int32(0)] * 2

        def _drain(bi):
            def _d(_, c2):
                pltpu.make_async_copy(
                    bufs[bi], out_hbm.at[pl.ds(tok0, _K)], osems[bi]
                ).wait()
                return c2
            lax.fori_loop(0, cnt[bi], _d, jnp.int32(0))

        pltpu.async_copy(table_hbm.at[idx_v.at[pl.ds(0, _K)]], bufs[0], gsem0)
        for c in range(NCHUNK):
            bi = c % 2
            ni = (c + 1) % 2
            buf = bufs[bi]
            coff = c * _K
            pltpu.make_async_copy(
                table_hbm.at[idx_v.at[pl.ds(0, _K)]], buf, gsems[bi]
            ).wait()
            if c + 1 < NCHUNK:
                _drain(ni)
                cnt[ni] = jnp.int32(0)
                pltpu.async_copy(
                    table_hbm.at[idx_v.at[pl.ds((c + 1) * _K, _K)]],
                    bufs[ni], gsems[ni],
                )
            pltpu.async_copy(
                buf, out_hbm.at[pl.ds(tok0 + coff, _K)], osems[bi]
            )
            ncopy = jnp.int32(1)
            for r in range(1, B):
                roff = r * SPAN + coff
                d0 = jnp.where(
                    idx_v[pl.ds(roff, _LANES)] == idx_v[pl.ds(coff, _LANES)],
                    zero, one,
                )
                d1 = jnp.where(
                    idx_v[pl.ds(roff + _LANES, _LANES)]
                    == idx_v[pl.ds(coff + _LANES, _LANES)],
                    zero, one,
                )
                same = jnp.sum(d0 + d1) == 0
                dst = out_hbm.at[pl.ds(r * S + tok0 + coff, _K)]

                @pl.when(same)
                def _reuse(buf=buf, dst=dst, bi=bi):
                    pltpu.async_copy(buf, dst, osems[bi])

                @pl.when(jnp.logical_not(same))
                def _regather(roff=roff, dst=dst):
                    pltpu.async_copy(
                        table_hbm.at[idx_v.at[pl.ds(roff, _K)]], bufb, msem
                    ).wait()
                    pltpu.sync_copy(bufb, dst)

                ncopy = ncopy + jnp.where(same, jnp.int32(1), jnp.int32(0))
            cnt[bi] = ncopy
        for bi in range(2):
            _drain(bi)

    return sc_kernel


def kernel(input, weights):
    B, S = input.shape
    _, D = weights.shape
    out = _build_sc_kernel(B, S, D)(input.reshape(-1), weights)
    return out.reshape(B, S, D)


# speedup vs baseline: 1.0022x; 1.0022x over previous
import functools

import jax
import jax.numpy as jnp
from jax import lax
from jax.experimental import pallas as pl
from jax.experimental.pallas import tpu as pltpu
from jax.experimental.pallas import tpu_sc as plsc

_PAD = 1
_LANES = 16
_NW = 32
_K = 32


@functools.lru_cache(maxsize=None)
def _build_sc_kernel(B, S, D):
    SPAN = S // _NW
    NCHUNK = SPAN // _K
    mesh = plsc.VectorSubcoreMesh(core_axis_name="c", subcore_axis_name="s")

    @functools.partial(
        pl.kernel,
        out_type=jax.ShapeDtypeStruct((B * S, D), jnp.float32),
        mesh=mesh,
        scratch_types=[
            pltpu.VMEM((S,), jnp.int32),
            pltpu.VMEM((S,), jnp.int32),
            pltpu.VMEM((B * SPAN,), jnp.int32),
            pltpu.VMEM((_K, D), jnp.float32),
            pltpu.VMEM((_K, D), jnp.float32),
            pltpu.VMEM((_K, D), jnp.float32),
            pltpu.SemaphoreType.DMA,
            pltpu.SemaphoreType.DMA,
            pltpu.SemaphoreType.DMA,
            pltpu.SemaphoreType.DMA,
            pltpu.SemaphoreType.DMA,
            pltpu.SemaphoreType.DMA,
        ],
        compiler_params=pltpu.CompilerParams(needs_layout_passes=False),
    )
    def sc_kernel(
        ids_hbm, table_hbm, out_hbm,
        ids0_v, ids1_v, idx_v, bufa0, bufa1, bufb,
        gsem0, gsem1, osem0, osem1, msem, isem,
    ):
        wid = lax.axis_index("s") * 2 + lax.axis_index("c")
        tok0 = wid * SPAN

        one = jnp.full((_LANES,), 1, jnp.int32)
        zero = jnp.full((_LANES,), 0, jnp.int32)
        pad_vec = jnp.full((_LANES,), _PAD, jnp.int32)

        _UNROLL = 8
        pltpu.async_copy(ids_hbm.at[pl.ds(0, S)], ids0_v, isem)
        for r in range(B):
            ids_v = ids0_v if r % 2 == 0 else ids1_v
            nxt_v = ids1_v if r % 2 == 0 else ids0_v
            pltpu.make_async_copy(ids_hbm.at[pl.ds(0, S)], ids_v, isem).wait()
            if r + 1 < B:
                pltpu.async_copy(
                    ids_hbm.at[pl.ds((r + 1) * S, S)], nxt_v, isem
                )

            def _cnt(j, acc, ids_v=ids_v):
                for u in range(_UNROLL):
                    v = ids_v[pl.ds((j * _UNROLL + u) * _LANES, _LANES)]
                    acc = acc + jnp.where(v != _PAD, one, zero)
                return acc
            accv = lax.fori_loop(
                0, wid * (SPAN // (_UNROLL * _LANES)), _cnt, zero
            )
            prefix = jnp.sum(accv)

            def _pos(j, run, ids_v=ids_v, r=r):
                v = ids_v[pl.ds(tok0 + j * _LANES, _LANES)]
                m = v != _PAD
                c = jnp.cumsum(jnp.where(m, one, zero))
                idx_v[pl.ds(r * SPAN + j * _LANES, _LANES)] = jnp.where(
                    m, c + run, pad_vec
                )
                return run + c[_LANES - 1]
            lax.fori_loop(0, SPAN // _LANES, _pos, prefix + jnp.int32(1))
            if r == 0:
                pltpu.async_copy(
                    table_hbm.at[idx_v.at[pl.ds(0, _K)]], bufa0, gsem0
                )

        bufs = (bufa0, bufa1)
        gsems = (gsem0, gsem1)
        osems = (osem0, osem1)
        cnt = [jnp.int32(0)] * 2

        def _drain(bi):
            def _d(_, c2):
                pltpu.make_async_copy(
                    bufs[bi], out_hbm.at[pl.ds(tok0, _K)], osems[bi]
                ).wait()
                return c2
            lax.fori_loop(0, cnt[bi], _d, jnp.int32(0))

        for c in range(NCHUNK):
            bi = c % 2
            ni = (c + 1) % 2
            buf = bufs[bi]
            coff = c * _K
            pltpu.make_async_copy(
                table_hbm.at[idx_v.at[pl.ds(0, _K)]], buf, gsems[bi]
            ).wait()
            if c + 1 < NCHUNK:
                _drain(ni)
                cnt[ni] = jnp.int32(0)
                pltpu.async_copy(
                    table_hbm.at[idx_v.at[pl.ds((c + 1) * _K, _K)]],
                    bufs[ni], gsems[ni],
                )
            pltpu.async_copy(
                buf, out_hbm.at[pl.ds(tok0 + coff, _K)], osems[bi]
            )
            ncopy = jnp.int32(1)
            for r in range(1, B):
                roff = r * SPAN + coff
                d0 = jnp.where(
                    idx_v[pl.ds(roff, _LANES)] == idx_v[pl.ds(coff, _LANES)],
                    zero, one,
                )
                d1 = jnp.where(
                    idx_v[pl.ds(roff + _LANES, _LANES)]
                    == idx_v[pl.ds(coff + _LANES, _LANES)],
                    zero, one,
                )
                same = jnp.sum(d0 + d1) == 0
                dst = out_hbm.at[pl.ds(r * S + tok0 + coff, _K)]

                @pl.when(same)
                def _reuse(buf=buf, dst=dst, bi=bi):
                    pltpu.async_copy(buf, dst, osems[bi])

                @pl.when(jnp.logical_not(same))
                def _regather(roff=roff, dst=dst):
                    pltpu.async_copy(
                        table_hbm.at[idx_v.at[pl.ds(roff, _K)]], bufb, msem
                    ).wait()
                    pltpu.sync_copy(bufb, dst)

                ncopy = ncopy + jnp.where(same, jnp.int32(1), jnp.int32(0))
            cnt[bi] = ncopy
        for bi in range(2):
            _drain(bi)

    return sc_kernel


def kernel(input, weights):
    B, S = input.shape
    _, D = weights.shape
    out = _build_sc_kernel(B, S, D)(input.reshape(-1), weights)
    return out.reshape(B, S, D)
